# baseline (speedup 1.0000x reference)
import functools

import jax
import jax.numpy as jnp
from jax import lax
from jax.experimental import pallas as pl
from jax.experimental.pallas import tpu as pltpu
from jax.experimental.pallas import tpu_sc as plsc

_D = 768
_NC = 2
_NS = 16
_NW = _NC * _NS


def _gather_rows(idx1, table, batch, seq):
    bat_per_w = batch // _NW
    mesh = plsc.VectorSubcoreMesh(core_axis_name="c", subcore_axis_name="s")

    @functools.partial(
        pl.kernel,
        mesh=mesh,
        out_type=jax.ShapeDtypeStruct((seq, batch, _D), jnp.float32),
        scratch_types=[
            pltpu.VMEM((seq * bat_per_w,), jnp.int32),
            pltpu.VMEM((bat_per_w, _D), jnp.float32),
            pltpu.VMEM((bat_per_w, _D), jnp.float32),
            pltpu.VMEM((bat_per_w, _D), jnp.float32),
            pltpu.VMEM((bat_per_w, _D), jnp.float32),
            pltpu.VMEM((bat_per_w, _D), jnp.float32),
            pltpu.SemaphoreType.DMA,
            pltpu.SemaphoreType.DMA,
        ],
    )
    def k(idx_hbm, table_hbm, out_hbm, idx_v, rows0_v, rows1_v, rows2_v,
          rows3_v, rows4_v, gsem, osem):
        wid = lax.axis_index("s") * _NC + lax.axis_index("c")
        base_b = wid * bat_per_w
        pltpu.sync_copy(
            idx_hbm.at[pl.ds(wid * seq * bat_per_w, seq * bat_per_w)], idx_v)

        def idx_slice(l):
            return idx_v.at[pl.ds(l * bat_per_w, bat_per_w)]

        bufs = (rows0_v, rows1_v, rows2_v, rows3_v, rows4_v)
        nb = len(bufs)
        gd = [None] * seq
        od = [None] * seq
        for m in range(nb - 1):
            gd[m] = pltpu.async_copy(
                table_hbm.at[idx_slice(m)], bufs[m % nb], gsem)
        for l in range(seq):
            gd[l].wait()
            od[l] = pltpu.async_copy(
                bufs[l % nb], out_hbm.at[l].at[pl.ds(base_b, bat_per_w)], osem)
            m = l + nb - 1
            if m < seq:
                if l >= 1:
                    od[l - 1].wait()
                gd[m] = pltpu.async_copy(
                    table_hbm.at[idx_slice(m)], bufs[m % nb], gsem)
        for l in range(max(0, seq - nb), seq):
            od[l].wait()

    return k(idx1, table)


def kernel(input_ids, table):
    b, s = input_ids.shape
    bat_per_w = b // _NW
    idx1 = (input_ids.astype(jnp.int32).T
            .reshape(s, _NW, bat_per_w)
            .transpose(1, 0, 2)
            .reshape(-1))
    out_t = _gather_rows(idx1, table, b, s)
    return jnp.transpose(out_t, (1, 0, 2))
